# baseline (speedup 1.0000x reference)
import functools

import jax
import jax.numpy as jnp
from jax import lax
from jax.experimental import pallas as pl
from jax.experimental.pallas import tpu as pltpu
from jax.experimental.pallas import tpu_sc as plsc

NC = 2
NS = 16
NW = NC * NS
LANES = 16
IDX_PER_STREAM = 128
HALF = 32
TW = 256


def _tree_sum(vals):
    while len(vals) > 1:
        nxt = []
        for i in range(0, len(vals) - 1, 2):
            nxt.append(vals[i] + vals[i + 1])
        if len(vals) % 2:
            nxt.append(vals[-1])
        vals = nxt
    return vals[0]


def _make_transpose_kernel(V, D):
    n_full = (V // TW)
    v_tail = V - n_full * TW
    base, extra = divmod(n_full, NW)
    mesh = plsc.VectorSubcoreMesh(core_axis_name="c", subcore_axis_name="s")

    @functools.partial(
        pl.kernel,
        mesh=mesh,
        out_type=jax.ShapeDtypeStruct((V * D,), jnp.float32),
        compiler_params=pltpu.CompilerParams(use_tc_tiling_on_sc=True,
                                             needs_layout_passes=False),
        scratch_types=[
            pltpu.VMEM((D, TW), jnp.float32),
            pltpu.VMEM((D, TW), jnp.float32),
            pltpu.VMEM((TW * D,), jnp.float32),
            pltpu.VMEM((TW * D,), jnp.float32),
            pltpu.VMEM((D, v_tail), jnp.float32),
            pltpu.VMEM((v_tail * D,), jnp.float32),
            pltpu.SemaphoreType.DMA,
            pltpu.SemaphoreType.DMA,
            pltpu.SemaphoreType.DMA,
        ],
    )
    def transpose_k(tabt_hbm, tailt_hbm, buf_hbm, slab0_v, slab1_v, stg0_v,
                    stg1_v, tail_v, tstg_v, sem0, sem1, out_sem):
        wid = lax.axis_index("s") * NC + lax.axis_index("c")
        n_i = base + jnp.where(wid < extra, 1, 0)
        sems = (sem0, sem1)
        slabs = (slab0_v, slab1_v)
        stgs = (stg0_v, stg1_v)
        lanes_i = lax.iota(jnp.int32, LANES)

        def slab_copy(i, par):
            g = wid + NW * i
            return pltpu.make_async_copy(
                tabt_hbm.at[:, pl.ds(g * TW, TW)], slabs[par], sems[par])

        def transpose_slab(src, dst, width):
            def v_body(vl, carry):
                for h in range(D // LANES):
                    d_idx = lanes_i + h * LANES
                    v_idx = jnp.broadcast_to(vl, (LANES,)).astype(jnp.int32)
                    vals = plsc.load_gather(src, [d_idx, v_idx])
                    dst[pl.ds(vl * D + h * LANES, LANES)] = vals
                return carry

            lax.fori_loop(0, width, v_body, 0, unroll=False)

        def flush(i, par):
            g = wid + NW * i
            return pltpu.make_async_copy(
                stgs[par], buf_hbm.at[pl.ds(g * TW * D, TW * D)], out_sem)

        @pl.when(n_i > 0)
        def _():
            slab_copy(0, 0).start()

            def chunk_body(i, carry):
                par = lax.rem(i, 2)

                @pl.when(i + 1 < n_i)
                def _():
                    @pl.when(par == 0)
                    def _():
                        slab_copy(i + 1, 1).start()

                    @pl.when(par == 1)
                    def _():
                        slab_copy(i + 1, 0).start()

                @pl.when(par == 0)
                def _():
                    slab_copy(i, 0).wait()

                    @pl.when(i >= 2)
                    def _():
                        flush(i - 2, 0).wait()

                    transpose_slab(slab0_v, stg0_v, TW)
                    flush(i, 0).start()

                @pl.when(par == 1)
                def _():
                    slab_copy(i, 1).wait()

                    @pl.when(i >= 2)
                    def _():
                        flush(i - 2, 1).wait()

                    transpose_slab(slab1_v, stg1_v, TW)
                    flush(i, 1).start()

                return carry

            lax.fori_loop(0, n_i, chunk_body, 0, unroll=False)

            last_par = lax.rem(n_i - 1, 2)
            for par in range(2):
                @pl.when(jnp.logical_and(n_i >= 2, last_par != par))
                def _():
                    flush(n_i - 2, par).wait()

                @pl.when(last_par == par)
                def _():
                    flush(n_i - 1, par).wait()

        if v_tail:
            @pl.when(wid == 2)
            def _():
                pltpu.sync_copy(tailt_hbm, tail_v)
                transpose_slab(tail_v, tstg_v, v_tail)
                pltpu.sync_copy(
                    tstg_v,
                    buf_hbm.at[pl.ds(n_full * TW * D, v_tail * D)])

    return transpose_k


def _make_gather_kernel(B, L, V, D, CH):
    b_per_w = B // NW
    n_idx = b_per_w * L
    n_chunks = b_per_w // CH
    rows_per_chunk = CH * L
    steps_per_chunk = rows_per_chunk // IDX_PER_STREAM
    inv_l = 1.0 / L

    mesh = plsc.VectorSubcoreMesh(core_axis_name="c", subcore_axis_name="s")

    @functools.partial(
        pl.kernel,
        mesh=mesh,
        out_type=jax.ShapeDtypeStruct((B, D), jnp.float32),
        compiler_params=pltpu.CompilerParams(use_tc_tiling_on_sc=False),
        scratch_types=[
            pltpu.VMEM((n_idx,), jnp.int32),
            pltpu.VMEM((n_idx,), jnp.int32),
            pltpu.VMEM((n_idx,), jnp.int32),
            pltpu.VMEM((2, rows_per_chunk, HALF), jnp.float32),
            pltpu.VMEM((2, rows_per_chunk, HALF), jnp.float32),
            pltpu.VMEM((2, CH, D), jnp.float32),
            pltpu.SemaphoreType.DMA,
            pltpu.SemaphoreType.DMA,
            pltpu.SemaphoreType.DMA,
        ],
    )
    def sc_kernel(ctx_hbm, table_hbm, out_hbm, ctx_raw, ia_v, ib_v,
                  rows_a, rows_b, ob_v, sem0, sem1, out_sem):
        wid = lax.axis_index("s") * NC + lax.axis_index("c")
        pltpu.sync_copy(ctx_hbm.at[wid], ctx_raw)
        sems = (sem0, sem1)

        def prep_body(k, carry):
            sl = pl.ds(k * LANES, LANES)
            v = ctx_raw[sl]
            a = v + v
            ia_v[sl] = a
            ib_v[sl] = a + 1
            return carry

        lax.fori_loop(0, n_idx // LANES, prep_body, 0, unroll=False)

        def gather_copies(c, par):
            for s in range(steps_per_chunk):
                off = c * rows_per_chunk + s * IDX_PER_STREAM
                dst = pl.ds(s * IDX_PER_STREAM, IDX_PER_STREAM)
                yield pltpu.make_async_copy(
                    table_hbm.at[ia_v.at[pl.ds(off, IDX_PER_STREAM)]],
                    rows_a.at[par].at[dst], sems[par])
                yield pltpu.make_async_copy(
                    table_hbm.at[ib_v.at[pl.ds(off, IDX_PER_STREAM)]],
                    rows_b.at[par].at[dst], sems[par])

        def fire(c, par):
            for cp in gather_copies(c, par):
                cp.start()

        def drain(c, par):
            for cp in gather_copies(c, par):
                cp.wait()

        def accumulate(c, par):
            @pl.when(c >= 2)
            def _():
                pltpu.make_async_copy(
                    ob_v.at[par], out_hbm.at[pl.ds(0, CH)], out_sem
                ).wait()

            def e_body(e, carry2):
                bs = e * L
                for half, rows in ((0, rows_a), (1, rows_b)):
                    for cg in range(HALF // LANES):
                        sl = pl.ds(cg * LANES, LANES)
                        vals = [rows[par, bs + j, sl] for j in range(L)]
                        osl = pl.ds((half * (HALF // LANES) + cg) * LANES,
                                    LANES)
                        ob_v[par, e, osl] = _tree_sum(vals) * inv_l
                return carry2

            lax.fori_loop(0, CH, e_body, 0, unroll=False)
            out_base = wid * b_per_w + c * CH
            pltpu.make_async_copy(
                ob_v.at[par], out_hbm.at[pl.ds(out_base, CH)], out_sem
            ).start()

        fire(0, 0)

        def pair_body(p, carry):
            c = p * 2
            fire(c + 1, 1)
            drain(c, 0)
            accumulate(c, 0)

            @pl.when(c + 2 < n_chunks)
            def _():
                fire(c + 2, 0)

            drain(c + 1, 1)
            accumulate(c + 1, 1)
            return carry

        lax.fori_loop(0, n_chunks // 2, pair_body, 0, unroll=False)
        for par in range(2):
            pltpu.make_async_copy(
                ob_v.at[par], out_hbm.at[pl.ds(0, CH)], out_sem
            ).wait()

    return sc_kernel


@jax.jit
def kernel(contexts, table):
    B, L = contexts.shape
    V, D = table.shape
    CH = 32
    assert (CH * L) % IDX_PER_STREAM == 0
    assert B % (NW * CH) == 0
    assert D == 2 * HALF
    tabt = table.T
    n_full = V // TW
    tailt = tabt[:, n_full * TW:]
    buf = _make_transpose_kernel(V, D)(tabt, tailt)
    tab32 = buf.reshape(V * 2, D // 2)
    ctx2 = contexts.reshape(NW, (B // NW) * L)
    return _make_gather_kernel(B, L, V, D, CH)(ctx2, tab32)


# speedup vs baseline: 1.2130x; 1.2130x over previous
import functools

import jax
import jax.numpy as jnp
from jax import lax
from jax.experimental import pallas as pl
from jax.experimental.pallas import tpu as pltpu
from jax.experimental.pallas import tpu_sc as plsc

NC = 2
NS = 16
NW = NC * NS
LANES = 16
IDX_PER_STREAM = 128
HALF = 32
TW = 256


def _tree_sum(vals):
    while len(vals) > 1:
        nxt = []
        for i in range(0, len(vals) - 1, 2):
            nxt.append(vals[i] + vals[i + 1])
        if len(vals) % 2:
            nxt.append(vals[-1])
        vals = nxt
    return vals[0]


def _make_transpose_kernel(V, D):
    n_full = (V // TW)
    v_tail = V - n_full * TW
    base, extra = divmod(n_full, NW)
    mesh = plsc.VectorSubcoreMesh(core_axis_name="c", subcore_axis_name="s")

    @functools.partial(
        pl.kernel,
        mesh=mesh,
        out_type=jax.ShapeDtypeStruct((V * D,), jnp.float32),
        compiler_params=pltpu.CompilerParams(use_tc_tiling_on_sc=True,
                                             needs_layout_passes=False),
        scratch_types=[
            pltpu.VMEM((D, TW), jnp.float32),
            pltpu.VMEM((D, TW), jnp.float32),
            pltpu.VMEM((TW * D,), jnp.float32),
            pltpu.VMEM((TW * D,), jnp.float32),
            pltpu.VMEM((D, v_tail), jnp.float32),
            pltpu.VMEM((v_tail * D,), jnp.float32),
            pltpu.SemaphoreType.DMA,
            pltpu.SemaphoreType.DMA,
            pltpu.SemaphoreType.DMA,
        ],
    )
    def transpose_k(tabt_hbm, tailt_hbm, buf_hbm, slab0_v, slab1_v, stg0_v,
                    stg1_v, tail_v, tstg_v, sem0, sem1, out_sem):
        wid = lax.axis_index("s") * NC + lax.axis_index("c")
        n_i = base + jnp.where(wid < extra, 1, 0)
        sems = (sem0, sem1)
        slabs = (slab0_v, slab1_v)
        stgs = (stg0_v, stg1_v)
        lanes_i = lax.iota(jnp.int32, LANES)

        def slab_copy(i, par):
            g = wid + NW * i
            return pltpu.make_async_copy(
                tabt_hbm.at[:, pl.ds(g * TW, TW)], slabs[par], sems[par])

        def transpose_slab(src, dst, width):
            lanes_d = lanes_i * D

            def k_body(k, carry):
                kb = k * (LANES * D)
                for d in range(D):
                    vals = src[d, pl.ds(k * LANES, LANES)]
                    plsc.store_scatter(dst, [lanes_d + (kb + d)], vals)
                return carry

            lax.fori_loop(0, width // LANES, k_body, 0, unroll=False)

        def flush(i, par):
            g = wid + NW * i
            return pltpu.make_async_copy(
                stgs[par], buf_hbm.at[pl.ds(g * TW * D, TW * D)], out_sem)

        @pl.when(n_i > 0)
        def _():
            slab_copy(0, 0).start()

            def chunk_body(i, carry):
                par = lax.rem(i, 2)

                @pl.when(i + 1 < n_i)
                def _():
                    @pl.when(par == 0)
                    def _():
                        slab_copy(i + 1, 1).start()

                    @pl.when(par == 1)
                    def _():
                        slab_copy(i + 1, 0).start()

                @pl.when(par == 0)
                def _():
                    slab_copy(i, 0).wait()

                    @pl.when(i >= 2)
                    def _():
                        flush(i - 2, 0).wait()

                    transpose_slab(slab0_v, stg0_v, TW)
                    flush(i, 0).start()

                @pl.when(par == 1)
                def _():
                    slab_copy(i, 1).wait()

                    @pl.when(i >= 2)
                    def _():
                        flush(i - 2, 1).wait()

                    transpose_slab(slab1_v, stg1_v, TW)
                    flush(i, 1).start()

                return carry

            lax.fori_loop(0, n_i, chunk_body, 0, unroll=False)

            last_par = lax.rem(n_i - 1, 2)
            for par in range(2):
                @pl.when(jnp.logical_and(n_i >= 2, last_par != par))
                def _():
                    flush(n_i - 2, par).wait()

                @pl.when(last_par == par)
                def _():
                    flush(n_i - 1, par).wait()

        if v_tail:
            @pl.when(wid == 2)
            def _():
                pltpu.sync_copy(tailt_hbm, tail_v)
                transpose_slab(tail_v, tstg_v, v_tail)
                pltpu.sync_copy(
                    tstg_v,
                    buf_hbm.at[pl.ds(n_full * TW * D, v_tail * D)])

    return transpose_k


def _make_gather_kernel(B, L, V, D, CH):
    b_per_w = B // NW
    n_idx = b_per_w * L
    n_chunks = b_per_w // CH
    rows_per_chunk = CH * L
    steps_per_chunk = rows_per_chunk // IDX_PER_STREAM
    inv_l = 1.0 / L

    mesh = plsc.VectorSubcoreMesh(core_axis_name="c", subcore_axis_name="s")

    @functools.partial(
        pl.kernel,
        mesh=mesh,
        out_type=jax.ShapeDtypeStruct((B, D), jnp.float32),
        compiler_params=pltpu.CompilerParams(use_tc_tiling_on_sc=False),
        scratch_types=[
            pltpu.VMEM((n_idx,), jnp.int32),
            pltpu.VMEM((n_idx,), jnp.int32),
            pltpu.VMEM((n_idx,), jnp.int32),
            pltpu.VMEM((2, rows_per_chunk, HALF), jnp.float32),
            pltpu.VMEM((2, rows_per_chunk, HALF), jnp.float32),
            pltpu.VMEM((2, CH, D), jnp.float32),
            pltpu.SemaphoreType.DMA,
            pltpu.SemaphoreType.DMA,
            pltpu.SemaphoreType.DMA,
        ],
    )
    def sc_kernel(ctx_hbm, table_hbm, out_hbm, ctx_raw, ia_v, ib_v,
                  rows_a, rows_b, ob_v, sem0, sem1, out_sem):
        wid = lax.axis_index("s") * NC + lax.axis_index("c")
        pltpu.sync_copy(ctx_hbm.at[wid], ctx_raw)
        sems = (sem0, sem1)

        def prep_body(k, carry):
            sl = pl.ds(k * LANES, LANES)
            v = ctx_raw[sl]
            a = v + v
            ia_v[sl] = a
            ib_v[sl] = a + 1
            return carry

        lax.fori_loop(0, n_idx // LANES, prep_body, 0, unroll=False)

        def gather_copies(c, par):
            for s in range(steps_per_chunk):
                off = c * rows_per_chunk + s * IDX_PER_STREAM
                dst = pl.ds(s * IDX_PER_STREAM, IDX_PER_STREAM)
                yield pltpu.make_async_copy(
                    table_hbm.at[ia_v.at[pl.ds(off, IDX_PER_STREAM)]],
                    rows_a.at[par].at[dst], sems[par])
                yield pltpu.make_async_copy(
                    table_hbm.at[ib_v.at[pl.ds(off, IDX_PER_STREAM)]],
                    rows_b.at[par].at[dst], sems[par])

        def fire(c, par):
            for cp in gather_copies(c, par):
                cp.start()

        def drain(c, par):
            for cp in gather_copies(c, par):
                cp.wait()

        def accumulate(c, par):
            @pl.when(c >= 2)
            def _():
                pltpu.make_async_copy(
                    ob_v.at[par], out_hbm.at[pl.ds(0, CH)], out_sem
                ).wait()

            def e_body(e, carry2):
                bs = e * L
                for half, rows in ((0, rows_a), (1, rows_b)):
                    for cg in range(HALF // LANES):
                        sl = pl.ds(cg * LANES, LANES)
                        vals = [rows[par, bs + j, sl] for j in range(L)]
                        osl = pl.ds((half * (HALF // LANES) + cg) * LANES,
                                    LANES)
                        ob_v[par, e, osl] = _tree_sum(vals) * inv_l
                return carry2

            lax.fori_loop(0, CH, e_body, 0, unroll=False)
            out_base = wid * b_per_w + c * CH
            pltpu.make_async_copy(
                ob_v.at[par], out_hbm.at[pl.ds(out_base, CH)], out_sem
            ).start()

        fire(0, 0)

        def pair_body(p, carry):
            c = p * 2
            fire(c + 1, 1)
            drain(c, 0)
            accumulate(c, 0)

            @pl.when(c + 2 < n_chunks)
            def _():
                fire(c + 2, 0)

            drain(c + 1, 1)
            accumulate(c + 1, 1)
            return carry

        lax.fori_loop(0, n_chunks // 2, pair_body, 0, unroll=False)
        for par in range(2):
            pltpu.make_async_copy(
                ob_v.at[par], out_hbm.at[pl.ds(0, CH)], out_sem
            ).wait()

    return sc_kernel


@jax.jit
def kernel(contexts, table):
    B, L = contexts.shape
    V, D = table.shape
    CH = 32
    assert (CH * L) % IDX_PER_STREAM == 0
    assert B % (NW * CH) == 0
    assert D == 2 * HALF
    tabt = table.T
    n_full = V // TW
    tailt = tabt[:, n_full * TW:]
    buf = _make_transpose_kernel(V, D)(tabt, tailt)
    tab32 = buf.reshape(V * 2, D // 2)
    ctx2 = contexts.reshape(NW, (B // NW) * L)
    return _make_gather_kernel(B, L, V, D, CH)(ctx2, tab32)


# speedup vs baseline: 4.4520x; 3.6703x over previous
import functools

import jax
import jax.numpy as jnp
from jax import lax
from jax.experimental import pallas as pl
from jax.experimental.pallas import tpu as pltpu
from jax.experimental.pallas import tpu_sc as plsc

NC = 2
NS = 16
NW = NC * NS
LANES = 16
IDX_PER_STREAM = 128
HALF = 32
TW = 256


def _tree_sum(vals):
    while len(vals) > 1:
        nxt = []
        for i in range(0, len(vals) - 1, 2):
            nxt.append(vals[i] + vals[i + 1])
        if len(vals) % 2:
            nxt.append(vals[-1])
        vals = nxt
    return vals[0]


def _make_transpose_kernel(V, D):
    n_full = (V // TW)
    v_tail = V - n_full * TW
    base, extra = divmod(n_full, NW)
    mesh = plsc.VectorSubcoreMesh(core_axis_name="c", subcore_axis_name="s")

    @functools.partial(
        pl.kernel,
        mesh=mesh,
        out_type=jax.ShapeDtypeStruct((V * D,), jnp.float32),
        compiler_params=pltpu.CompilerParams(use_tc_tiling_on_sc=True,
                                             needs_layout_passes=False),
        scratch_types=[
            pltpu.VMEM((D, TW), jnp.float32),
            pltpu.VMEM((D, TW), jnp.float32),
            pltpu.VMEM((TW * D,), jnp.float32),
            pltpu.VMEM((TW * D,), jnp.float32),
            pltpu.VMEM((D, v_tail), jnp.float32),
            pltpu.VMEM((v_tail * D,), jnp.float32),
            pltpu.VMEM((LANES * LANES,), jnp.int32),
            pltpu.VMEM((LANES * LANES,), jnp.int32),
            pltpu.SemaphoreType.DMA,
            pltpu.SemaphoreType.DMA,
            pltpu.SemaphoreType.DMA,
        ],
    )
    def transpose_k(tabt_hbm, tailt_hbm, buf_hbm, slab0_v, slab1_v, stg0_v,
                    stg1_v, tail_v, tstg_v, wrapv, dcolv, sem0, sem1,
                    out_sem):
        wid = lax.axis_index("s") * NC + lax.axis_index("c")
        n_i = base + jnp.where(wid < extra, 1, 0)
        sems = (sem0, sem1)
        slabs = (slab0_v, slab1_v)
        stgs = (stg0_v, stg1_v)
        lanes_i = lax.iota(jnp.int32, LANES)

        def slab_copy(i, par):
            g = wid + NW * i
            return pltpu.make_async_copy(
                tabt_hbm.at[:, pl.ds(g * TW, TW)], slabs[par], sems[par])

        for t in range(LANES):
            w = lax.rem(lanes_i + t, LANES)
            wrapv[pl.ds(t * LANES, LANES)] = w
            dcolv[pl.ds(t * LANES, LANES)] = w * D + lanes_i
        dvecs = [lanes_i + db * LANES for db in range(D // LANES)]

        def transpose_slab(src, dst, width):
            for t in range(LANES):
                wrap = wrapv[pl.ds(t * LANES, LANES)]
                dcol = dcolv[pl.ds(t * LANES, LANES)]

                def vb_body(vb, carry):
                    v0 = vb * LANES
                    vidx = wrap + v0
                    vals = [plsc.load_gather(src, [dvecs[db], vidx])
                            for db in range(D // LANES)]
                    vbase = v0 * D
                    for db in range(D // LANES):
                        plsc.store_scatter(
                            dst, [dcol + (vbase + db * LANES)], vals[db])
                    return carry

                lax.fori_loop(0, width // LANES, vb_body, 0, unroll=2)

        def flush(i, par):
            g = wid + NW * i
            return pltpu.make_async_copy(
                stgs[par], buf_hbm.at[pl.ds(g * TW * D, TW * D)], out_sem)

        @pl.when(n_i > 0)
        def _():
            slab_copy(0, 0).start()

            def chunk_body(i, carry):
                par = lax.rem(i, 2)

                @pl.when(i + 1 < n_i)
                def _():
                    @pl.when(par == 0)
                    def _():
                        slab_copy(i + 1, 1).start()

                    @pl.when(par == 1)
                    def _():
                        slab_copy(i + 1, 0).start()

                @pl.when(par == 0)
                def _():
                    slab_copy(i, 0).wait()

                    @pl.when(i >= 2)
                    def _():
                        flush(i - 2, 0).wait()

                    transpose_slab(slab0_v, stg0_v, TW)
                    flush(i, 0).start()

                @pl.when(par == 1)
                def _():
                    slab_copy(i, 1).wait()

                    @pl.when(i >= 2)
                    def _():
                        flush(i - 2, 1).wait()

                    transpose_slab(slab1_v, stg1_v, TW)
                    flush(i, 1).start()

                return carry

            lax.fori_loop(0, n_i, chunk_body, 0, unroll=False)

            last_par = lax.rem(n_i - 1, 2)
            for par in range(2):
                @pl.when(jnp.logical_and(n_i >= 2, last_par != par))
                def _():
                    flush(n_i - 2, par).wait()

                @pl.when(last_par == par)
                def _():
                    flush(n_i - 1, par).wait()

        if v_tail:
            @pl.when(wid == 2)
            def _():
                pltpu.sync_copy(tailt_hbm, tail_v)
                transpose_slab(tail_v, tstg_v, v_tail)
                pltpu.sync_copy(
                    tstg_v,
                    buf_hbm.at[pl.ds(n_full * TW * D, v_tail * D)])

    return transpose_k


def _make_gather_kernel(B, L, V, D, CH):
    b_per_w = B // NW
    n_idx = b_per_w * L
    n_chunks = b_per_w // CH
    rows_per_chunk = CH * L
    steps_per_chunk = rows_per_chunk // IDX_PER_STREAM
    inv_l = 1.0 / L

    mesh = plsc.VectorSubcoreMesh(core_axis_name="c", subcore_axis_name="s")

    @functools.partial(
        pl.kernel,
        mesh=mesh,
        out_type=jax.ShapeDtypeStruct((B, D), jnp.float32),
        compiler_params=pltpu.CompilerParams(use_tc_tiling_on_sc=False),
        scratch_types=[
            pltpu.VMEM((n_idx,), jnp.int32),
            pltpu.VMEM((n_idx,), jnp.int32),
            pltpu.VMEM((n_idx,), jnp.int32),
            pltpu.VMEM((2, rows_per_chunk, HALF), jnp.float32),
            pltpu.VMEM((2, rows_per_chunk, HALF), jnp.float32),
            pltpu.VMEM((2, CH, D), jnp.float32),
            pltpu.SemaphoreType.DMA,
            pltpu.SemaphoreType.DMA,
            pltpu.SemaphoreType.DMA,
        ],
    )
    def sc_kernel(ctx_hbm, table_hbm, out_hbm, ctx_raw, ia_v, ib_v,
                  rows_a, rows_b, ob_v, sem0, sem1, out_sem):
        wid = lax.axis_index("s") * NC + lax.axis_index("c")
        pltpu.sync_copy(ctx_hbm.at[wid], ctx_raw)
        sems = (sem0, sem1)

        def prep_body(k, carry):
            sl = pl.ds(k * LANES, LANES)
            v = ctx_raw[sl]
            a = v + v
            ia_v[sl] = a
            ib_v[sl] = a + 1
            return carry

        lax.fori_loop(0, n_idx // LANES, prep_body, 0, unroll=False)

        def gather_copies(c, par):
            for s in range(steps_per_chunk):
                off = c * rows_per_chunk + s * IDX_PER_STREAM
                dst = pl.ds(s * IDX_PER_STREAM, IDX_PER_STREAM)
                yield pltpu.make_async_copy(
                    table_hbm.at[ia_v.at[pl.ds(off, IDX_PER_STREAM)]],
                    rows_a.at[par].at[dst], sems[par])
                yield pltpu.make_async_copy(
                    table_hbm.at[ib_v.at[pl.ds(off, IDX_PER_STREAM)]],
                    rows_b.at[par].at[dst], sems[par])

        def fire(c, par):
            for cp in gather_copies(c, par):
                cp.start()

        def drain(c, par):
            for cp in gather_copies(c, par):
                cp.wait()

        def accumulate(c, par):
            @pl.when(c >= 2)
            def _():
                pltpu.make_async_copy(
                    ob_v.at[par], out_hbm.at[pl.ds(0, CH)], out_sem
                ).wait()

            def e_body(e, carry2):
                bs = e * L
                for half, rows in ((0, rows_a), (1, rows_b)):
                    for cg in range(HALF // LANES):
                        sl = pl.ds(cg * LANES, LANES)
                        vals = [rows[par, bs + j, sl] for j in range(L)]
                        osl = pl.ds((half * (HALF // LANES) + cg) * LANES,
                                    LANES)
                        ob_v[par, e, osl] = _tree_sum(vals) * inv_l
                return carry2

            lax.fori_loop(0, CH, e_body, 0, unroll=False)
            out_base = wid * b_per_w + c * CH
            pltpu.make_async_copy(
                ob_v.at[par], out_hbm.at[pl.ds(out_base, CH)], out_sem
            ).start()

        fire(0, 0)

        def pair_body(p, carry):
            c = p * 2
            fire(c + 1, 1)
            drain(c, 0)
            accumulate(c, 0)

            @pl.when(c + 2 < n_chunks)
            def _():
                fire(c + 2, 0)

            drain(c + 1, 1)
            accumulate(c + 1, 1)
            return carry

        lax.fori_loop(0, n_chunks // 2, pair_body, 0, unroll=False)
        for par in range(2):
            pltpu.make_async_copy(
                ob_v.at[par], out_hbm.at[pl.ds(0, CH)], out_sem
            ).wait()

    return sc_kernel


@jax.jit
def kernel(contexts, table):
    B, L = contexts.shape
    V, D = table.shape
    CH = 32
    assert (CH * L) % IDX_PER_STREAM == 0
    assert B % (NW * CH) == 0
    assert D == 2 * HALF
    tabt = table.T
    n_full = V // TW
    tailt = tabt[:, n_full * TW:]
    buf = _make_transpose_kernel(V, D)(tabt, tailt)
    tab32 = buf.reshape(V * 2, D // 2)
    ctx2 = contexts.reshape(NW, (B // NW) * L)
    return _make_gather_kernel(B, L, V, D, CH)(ctx2, tab32)
